# baseline (speedup 1.0000x reference)
import functools
import math

import jax
import jax.numpy as jnp
from jax import lax
from jax.experimental import pallas as pl
from jax.experimental.pallas import tpu as pltpu
from jax.experimental.pallas import tpu_sc as plsc

ALPHA = 0.1
B, M, C = 1024, 128, 384
NC, NS = 2, 16
NW = NC * NS
BPW = B // NW


def _compute_body(x_ref, mm_ref, z_ref, kl_ref, idx_ref, rows_ref):
    X = x_ref[...]
    MM = mm_ref[...]
    sims = jax.lax.dot_general(X, MM, (((1,), (1,)), ((), ())),
                               preferred_element_type=jnp.float32)
    closest = jnp.argmax(sims, axis=1)
    onehot = (jax.lax.broadcasted_iota(jnp.int32, (B, M), 1)
              == closest[:, None])
    oh_f = onehot.astype(jnp.float32)
    gathered = jax.lax.dot_general(oh_f, MM, (((1,), (0,)), ((), ())),
                                   preferred_element_type=jnp.float32)
    diff = X - gathered
    delta = ALPHA * diff
    xsq = jnp.sum(X * X, axis=1)
    s_at = jnp.sum(sims * oh_f, axis=1)
    corr = (1.0 - ALPHA) * s_at + ALPHA * xsq
    scores = jnp.where(onehot, corr[:, None], sims) * (1.0 / math.sqrt(C))
    smax = jnp.max(scores, axis=1, keepdims=True)
    e = jnp.exp(scores - smax)
    w = e / jnp.sum(e, axis=1, keepdims=True)
    z = jax.lax.dot_general(w, MM, (((1,), (0,)), ((), ())),
                            preferred_element_type=jnp.float32)
    w_at = jnp.sum(w * oh_f, axis=1)
    z = z + w_at[:, None] * delta
    z_ref[...] = z
    kl_ref[...] = 0.5 * (jnp.sum(diff * diff, axis=1)
                         + jnp.sum((z - X) ** 2, axis=1))
    idx_ref[...] = closest + M * jax.lax.broadcasted_iota(jnp.int32, (B,), 0)
    rows_ref[...] = gathered + delta


def _sc_writer_body(mm_hbm, rows_hbm, idx_hbm, post_hbm,
                    mm_v, rows_v, idx_v, bsem, ssem):
    wid = lax.axis_index("s") * NC + lax.axis_index("c")
    base = wid * BPW
    pltpu.sync_copy(mm_hbm, mm_v)
    pltpu.sync_copy(rows_hbm.at[pl.ds(base, BPW)], rows_v)
    pltpu.sync_copy(idx_hbm.at[pl.ds(base, BPW)], idx_v)
    for b in range(BPW):
        pltpu.async_copy(mm_v, post_hbm.at[pl.ds((base + b) * M, M)], bsem)
    for b in range(BPW):
        pltpu.make_async_copy(mm_v, post_hbm.at[pl.ds((base + b) * M, M)],
                              bsem).wait()
    pltpu.async_copy(rows_v, post_hbm.at[idx_v], ssem).wait()


_sc_writer = functools.partial(
    pl.kernel,
    out_type=jax.ShapeDtypeStruct((B * M, C), jnp.float32),
    mesh=plsc.VectorSubcoreMesh(core_axis_name="c", subcore_axis_name="s"),
    scratch_types=[
        pltpu.VMEM((M, C), jnp.float32),
        pltpu.VMEM((BPW, C), jnp.float32),
        pltpu.VMEM((BPW,), jnp.int32),
        pltpu.SemaphoreType.DMA,
        pltpu.SemaphoreType.DMA,
    ],
)(_sc_writer_body)


def kernel(input_encoded, memory_mean, memory_logvar):
    del memory_logvar

    z, kl, flat_idx, new_rows = pl.pallas_call(
        _compute_body,
        out_shape=[
            jax.ShapeDtypeStruct((B, C), jnp.float32),
            jax.ShapeDtypeStruct((B,), jnp.float32),
            jax.ShapeDtypeStruct((B,), jnp.int32),
            jax.ShapeDtypeStruct((B, C), jnp.float32),
        ],
    )(input_encoded, memory_mean)

    post_flat = _sc_writer(memory_mean, new_rows, flat_idx)
    posterior = post_flat.reshape(B, M, C)

    return z, posterior, kl
